# baseline (speedup 1.0000x reference)
import functools

import jax
import jax.numpy as jnp
from jax import lax
from jax.experimental import pallas as pl
from jax.experimental.pallas import tpu as pltpu
from jax.experimental.pallas import tpu_sc as plsc

B, N, D = 4, 10000, 128
E = 320000
NC, NS = 2, 16
EPT = E // NS
CH = 80
CHH = CH // 2
NCH = EPT // CH
SUP = 2000
CPS = SUP // CH
NSUP = EPT // SUP
WB = 624
TAIL = N - NS * WB
ZROWS = 48


def _tc_matmul(x2d, w_stacked):
    rb = 1000
    grid = (2, (B * N) // rb)

    def mm_kernel(x_ref, w_ref, y_ref):
        y_ref[0] = jnp.dot(x_ref[...], w_ref[0],
                           preferred_element_type=jnp.float32)

    return pl.pallas_call(
        mm_kernel,
        grid=grid,
        in_specs=[
            pl.BlockSpec((rb, D), lambda k, i: (i, 0)),
            pl.BlockSpec((1, D, D), lambda k, i: (k, 0, 0)),
        ],
        out_specs=pl.BlockSpec((1, rb, D), lambda k, i: (k, i, 0)),
        out_shape=jax.ShapeDtypeStruct((2, B * N, D), jnp.float32),
    )(x2d, w_stacked)


def _sc_spmm(y_planes, src, dst, w):
    mesh = plsc.VectorSubcoreMesh(core_axis_name="c", subcore_axis_name="s")

    @functools.partial(
        pl.kernel,
        out_type=jax.ShapeDtypeStruct((2 * B, N, D), jnp.float32),
        mesh=mesh,
        scratch_types=[
            pltpu.VMEM((2 * SUP,), jnp.int32),
            pltpu.VMEM((2 * SUP,), jnp.int32),
            pltpu.VMEM((2 * SUP,), jnp.float32),
            [pltpu.VMEM((CH, D), jnp.float32) for _ in range(3)],
            [pltpu.VMEM((1, CH), jnp.int32) for _ in range(3)],
            pltpu.VMEM((ZROWS, D), jnp.float32),
            pltpu.VMEM_SHARED((N, D), jnp.float32),
            [pltpu.SemaphoreType.DMA for _ in range(3)],
            [pltpu.SemaphoreType.DMA for _ in range(3)],
            pltpu.SemaphoreType.DMA,
        ],
    )
    def sc_kernel(y_hbm, src_hbm, dst_hbm, w_hbm, out_hbm,
                  esrc, edst, ew, rows, dst2, z_v, acc_sh, gsem, ssem, esem):
        c = lax.axis_index("c")
        s = lax.axis_index("s")
        ebase = c * E + s * EPT

        def zrow(i, carry):
            for r in range(D // 16):
                z_v[i, pl.ds(r * 16, 16)] = jnp.zeros((16,), jnp.float32)
            return carry

        lax.fori_loop(0, ZROWS, zrow, 0)

        def wait_gather(j):
            pltpu.make_async_copy(
                y_hbm.at[0, pl.ds(0, CH)], rows[j], gsem[j]).wait()

        def issue_scatter(j):
            pltpu.async_copy(rows[j], acc_sh.at[dst2[j].at[0]], ssem[j],
                             add=True)

        def wait_scatter(j):
            pltpu.make_async_copy(
                rows[j], acc_sh.at[pl.ds(0, CH)], ssem[j]).wait()

        def stage_super(u, sync):
            poff = (u % 2) * SUP
            hoff = ebase + u * SUP
            for hbm, ref in ((src_hbm, esrc), (dst_hbm, edst), (w_hbm, ew)):
                pltpu.async_copy(hbm.at[pl.ds(hoff, SUP)],
                                 ref.at[pl.ds(poff, SUP)], esem)
            if sync:
                drain_super()

        def drain_super():
            for hbm, ref in ((src_hbm, esrc), (dst_hbm, edst), (w_hbm, ew)):
                pltpu.make_async_copy(hbm.at[pl.ds(ebase, SUP)],
                                      ref.at[pl.ds(0, SUP)], esem).wait()

        def scale(i, j):
            u = i // CPS
            woff = (u % 2) * SUP + (i % CPS) * CH

            def grp(g, carry):
                w16 = ew[pl.ds(woff + g * 16, 16)]
                for jj in range(16):
                    we = w16[jj]
                    e = g * 16 + jj
                    for r in range(D // 16):
                        sl = pl.ds(r * 16, 16)
                        rows[j][e, sl] = rows[j][e, sl] * we
                return carry

            lax.fori_loop(0, CH // 16, grp, 0)

        def do_plane(b, carry):
            plane = c * B + b

            def prep_stage(inext, jn):
                un = inext // CPS
                ln = inext % CPS
                poff = (un % 2) * SUP

                @pl.when(jnp.logical_and(ln == 0, un >= 1))
                def _drain():
                    drain_super()

                @pl.when(jnp.logical_and(ln == 1, un < NSUP - 1))
                def _prefetch():
                    stage_super(un + 1, False)

                eoff = poff + ln * CH
                for r in range(CH // 16):
                    dst2[jn][0, pl.ds(r * 16, 16)] = (
                        edst[pl.ds(eoff + r * 16, 16)])
                pltpu.async_copy(
                    y_hbm.at[plane].at[esrc.at[pl.ds(eoff, CH)]],
                    rows[jn], gsem[jn])

            for i in range(WB // ZROWS):
                pltpu.sync_copy(
                    z_v, acc_sh.at[pl.ds(s * WB + i * ZROWS, ZROWS)])

            @pl.when(s == NS - 1)
            def _zero_tail():
                pltpu.sync_copy(z_v.at[pl.ds(0, TAIL)],
                                acc_sh.at[pl.ds(NS * WB, TAIL)])

            stage_super(0, True)
            plsc.subcore_barrier()

            prep_stage(0, 0)

            def iter3(h, carry2):
                i0 = 3 * h

                @pl.when(h > 0)
                def _c0():
                    wait_scatter(1)

                prep_stage(i0 + 1, 1)
                wait_gather(0)
                scale(i0, 0)
                issue_scatter(0)

                @pl.when(h > 0)
                def _c1():
                    wait_scatter(2)

                prep_stage(i0 + 2, 2)
                wait_gather(1)
                scale(i0 + 1, 1)
                issue_scatter(1)

                wait_scatter(0)
                prep_stage(i0 + 3, 0)
                wait_gather(2)
                scale(i0 + 2, 2)
                issue_scatter(2)
                return carry2

            lax.fori_loop(0, (NCH - 1) // 3, iter3, 0)
            wait_gather(0)
            scale(NCH - 1, 0)
            issue_scatter(0)
            wait_scatter(0)
            wait_scatter(1)
            wait_scatter(2)
            plsc.subcore_barrier()

            pltpu.sync_copy(
                acc_sh.at[pl.ds(s * WB, WB)],
                out_hbm.at[plane].at[pl.ds(s * WB, WB)])

            @pl.when(s == NS - 1)
            def _write_tail():
                pltpu.sync_copy(
                    acc_sh.at[pl.ds(NS * WB, TAIL)],
                    out_hbm.at[plane].at[pl.ds(NS * WB, TAIL)])

            plsc.subcore_barrier()
            return carry

        lax.fori_loop(0, B, do_plane, 0)

    return sc_kernel(y_planes, src, dst, w)


def kernel(inputs, edge_index0, edge_weight0, edge_index1, edge_weight1,
           W0, W1):
    x2d = inputs.reshape(B * N, D)
    w_stacked = jnp.stack([W0, W1])
    y = _tc_matmul(x2d, w_stacked).reshape(2 * B, N, D)
    src = jnp.concatenate([edge_index0[1], edge_index1[1]])
    dst = jnp.concatenate([edge_index0[0], edge_index1[0]])
    w = jnp.concatenate([edge_weight0, edge_weight1])
    out = _sc_spmm(y, src, dst, w)
    return out.reshape(2, B, N, D).transpose(1, 2, 0, 3).reshape(B, N, 2 * D)
